# baseline (speedup 1.0000x reference)
import functools

import jax
import jax.numpy as jnp
from jax import lax
from jax.experimental import pallas as pl
from jax.experimental.pallas import tpu as pltpu
from jax.experimental.pallas import tpu_sc as plsc

_CH = 4
_NB = 7
_LA = 3

_ROWS = 128


def _make_sc_copy(C, G, D, T, NC, NS):
    NW = NC * NS
    rows = C // NW
    nch = rows // _CH
    mesh = plsc.VectorSubcoreMesh(core_axis_name="c", subcore_axis_name="s")

    @functools.partial(
        pl.kernel,
        mesh=mesh,
        out_type=jax.ShapeDtypeStruct((C, G, D), jnp.float32),
        scratch_types=(
            [pltpu.VMEM((16,), jnp.int32)]
            + [pltpu.VMEM((_CH, G, D), jnp.float32) for _ in range(_NB)]
            + [pltpu.SemaphoreType.DMA for _ in range(2 * _NB)]
        ),
    )
    def sc_copy(seq_hbm, vb, nv, ov, seq_v, *bufs_sems):
        bufs = bufs_sems[:_NB]
        gsems = bufs_sems[_NB : 2 * _NB]
        ssems = bufs_sems[2 * _NB :]

        wid = lax.axis_index("s") * NC + lax.axis_index("c")
        base = wid * rows
        pltpu.sync_copy(seq_hbm, seq_v)
        seq = seq_v[...][0]

        def gather(c, b):
            r = base + c * _CH
            in_new = jnp.logical_and(r >= seq, r < seq + T)

            @pl.when(in_new)
            def _():
                pltpu.make_async_copy(
                    nv.at[pl.ds(r - seq, _CH)], bufs[b], gsems[b]
                ).start()

            @pl.when(jnp.logical_not(in_new))
            def _():
                pltpu.make_async_copy(
                    vb.at[pl.ds(r, _CH)], bufs[b], gsems[b]
                ).start()

        def wait_gather(b):
            pltpu.make_async_copy(vb.at[pl.ds(0, _CH)], bufs[b], gsems[b]).wait()

        def scatter(c, b):
            r = base + c * _CH
            pltpu.make_async_copy(bufs[b], ov.at[pl.ds(r, _CH)], ssems[b]).start()

        def wait_scatter(c, b):
            r = base + c * _CH
            pltpu.make_async_copy(bufs[b], ov.at[pl.ds(r, _CH)], ssems[b]).wait()

        for k in range(min(_LA + 1, nch)):
            gather(k, k % _NB)
        for j in range(nch):
            b = j % _NB
            nxt = j + _LA + 1
            if nxt < nch:
                bn = nxt % _NB
                if nxt >= _NB:
                    wait_scatter(nxt - _NB, bn)
                gather(nxt, bn)
            wait_gather(b)
            scatter(j, b)
        for j in range(max(nch - _NB, 0), nch):
            wait_scatter(j, j % _NB)

    return sc_copy


def _tc_copy(seqb, kb, nk, C, G, D, T):
    nb = C // _ROWS
    tb = T // _ROWS

    def body(seqb_ref, kb_ref, nk_ref, ok_ref):
        i = pl.program_id(0)
        sb = seqb_ref[0]
        use_new = jnp.logical_and(i >= sb, i < sb + tb)

        @pl.when(use_new)
        def _():
            ok_ref[...] = nk_ref[...]

        @pl.when(jnp.logical_not(use_new))
        def _():
            ok_ref[...] = kb_ref[...]

    def buf_map(i, seqb_ref):
        sb = seqb_ref[0]
        in_new = jnp.logical_and(i >= sb, i < sb + tb)
        return (jnp.where(in_new, jnp.maximum(sb - 1, 0), i), 0, 0)

    def new_map(i, seqb_ref):
        sb = seqb_ref[0]
        return (jnp.clip(i - sb, 0, tb - 1), 0, 0)

    blk = (_ROWS, G, D)
    grid_spec = pltpu.PrefetchScalarGridSpec(
        num_scalar_prefetch=1,
        grid=(nb,),
        in_specs=[pl.BlockSpec(blk, buf_map), pl.BlockSpec(blk, new_map)],
        out_specs=pl.BlockSpec(blk, lambda i, s: (i, 0, 0)),
    )
    return pl.pallas_call(
        body,
        grid_spec=grid_spec,
        out_shape=jax.ShapeDtypeStruct((C, G, D), jnp.float32),
    )(seqb, kb, nk)


def kernel(keys_buffer, values_buffer, new_keys, new_values, sequence_length):
    C, G, D = keys_buffer.shape
    T = new_keys.shape[0]
    seq = jnp.asarray(sequence_length, jnp.int32)

    info = plsc.get_sparse_core_info()
    NC, NS = info.num_cores, info.num_subcores

    seq16 = jnp.full((16,), seq, jnp.int32)
    ov = _make_sc_copy(C, G, D, T, NC, NS)(seq16, values_buffer, new_values)
    ok = _tc_copy((seq // _ROWS).reshape(1), keys_buffer, new_keys, C, G, D, T)

    return ((seq + T).astype(jnp.int32), ok, ov)
